# baseline (speedup 1.0000x reference)
import jax
import jax.numpy as jnp
from jax.experimental import pallas as pl
from jax.experimental.pallas import tpu as pltpu

T = 2048
D = 1024
E = 64
K = 8
F_MOE = 1408
F_SH = 2816
CAP = 512

_BF = jnp.bfloat16
_F32 = jnp.float32

_TCH = 512
_FCH = (768, 640)


def _routing_kernel(x_ref, gw_ref, gates_ref, rank_ref):
    x = x_ref[...]
    logits = jnp.dot(x, gw_ref[...], precision=jax.lax.Precision.HIGHEST)
    m = jnp.max(logits, axis=1, keepdims=True)
    p = jnp.exp(logits - m)
    p = p / jnp.sum(p, axis=1, keepdims=True)

    work = p
    kth = None
    for _ in range(K):
        kth = jnp.max(work, axis=1, keepdims=True)
        work = jnp.where(work == kth, -1.0, work)
    gates = jnp.where(p >= kth, p, 0.0)

    maskb = (gates > 0.0).astype(_BF)
    blk = 256
    row_i = jax.lax.broadcasted_iota(jnp.int32, (blk, blk), 0)
    col_i = jax.lax.broadcasted_iota(jnp.int32, (blk, blk), 1)
    ltri = (col_i <= row_i).astype(_BF)
    carry = jnp.zeros((1, E), dtype=_F32)
    chunks = []
    for c in range(T // blk):
        mc = maskb[c * blk:(c + 1) * blk, :]
        rc = jnp.dot(ltri, mc, preferred_element_type=_F32) + carry
        carry = carry + jnp.sum(mc.astype(_F32), axis=0, keepdims=True)
        chunks.append(rc)
    rank = jnp.concatenate(chunks, axis=0)
    rank = jnp.where(gates > 0.0, rank, 0.0)

    gates_ref[...] = gates
    rank_ref[...] = rank


def _rank_col(rank_ref, ee):
    lane = jax.lax.broadcasted_iota(jnp.int32, (1, E), 1)
    sel = lane == ee
    col = jnp.sum(jnp.where(sel, rank_ref[...], 0.0), axis=1, keepdims=True)
    return col.astype(jnp.int32)


def _gather_xe(xb_ref, rank_ref, ee):
    r_i = _rank_col(rank_ref, ee)
    cap_i = jax.lax.broadcasted_iota(jnp.int32, (_TCH, CAP), 1) + 1
    xe = jnp.zeros((CAP, D), dtype=_F32)
    for tb in range(T // _TCH):
        sl = slice(tb * _TCH, (tb + 1) * _TCH)
        ptc = (r_i[sl] == cap_i).astype(_BF)
        xe = xe + jax.lax.dot_general(
            ptc, xb_ref[sl], (((0,), (0,)), ((), ())),
            preferred_element_type=_F32)
    return xe.astype(_BF)


def _expert_kernel(xb_ref, gates_ref, rank_ref, w1_ref, w3_ref, w2_ref,
                   out_ref, xe_buf):
    e = pl.program_id(0)
    lane = jax.lax.broadcasted_iota(jnp.int32, (1, E), 1)
    sel = lane == e
    g_col = jnp.sum(jnp.where(sel, gates_ref[...], 0.0), axis=1, keepdims=True)
    r_i = _rank_col(rank_ref, e)

    cap_i = jax.lax.broadcasted_iota(jnp.int32, (_TCH, CAP), 1) + 1

    par = jax.lax.rem(e, 2)

    @pl.when(e == 0)
    def _():
        xe_buf[0] = _gather_xe(xb_ref, rank_ref, 0)

    xe = xe_buf[par]

    @pl.when(e + 1 < E)
    def _():
        xe_buf[1 - par] = _gather_xe(xb_ref, rank_ref, e + 1)

    y = jnp.zeros((CAP, D), dtype=_F32)
    lo = 0
    for sz in _FCH:
        w1h = w1_ref[0, :, lo:lo + sz].astype(_BF)
        a = jnp.dot(xe, w1h, preferred_element_type=_F32)
        w3h = w3_ref[0, :, lo:lo + sz].astype(_BF)
        b = jnp.dot(xe, w3h, preferred_element_type=_F32)
        h = (a * jax.nn.sigmoid(a) * b).astype(_BF)
        w2h = w2_ref[0, lo:lo + sz, :].astype(_BF)
        y = y + jnp.dot(h, w2h, preferred_element_type=_F32)
        lo += sz
    yb = y.astype(_BF)

    for tb in range(T // _TCH):
        sl = slice(tb * _TCH, (tb + 1) * _TCH)
        ptc = (r_i[sl] == cap_i).astype(_BF)
        contrib = jnp.dot(ptc, yb, preferred_element_type=_F32) * g_col[sl]

        @pl.when(e == 0)
        def _(sl=sl, contrib=contrib):
            out_ref[sl] = contrib

        @pl.when(e != 0)
        def _(sl=sl, contrib=contrib):
            out_ref[sl] = out_ref[sl] + contrib


def _shared_kernel(xb_ref, routed_ref, sw1_ref, sw3_ref, sw2_ref, sgw_ref,
                   out_ref, acc_ref):
    j = pl.program_id(1)
    nj = pl.num_programs(1)
    xb = xb_ref[...]
    a = jnp.dot(xb, sw1_ref[...].astype(_BF), preferred_element_type=_F32)
    b = jnp.dot(xb, sw3_ref[...].astype(_BF), preferred_element_type=_F32)
    h = (a * jax.nn.sigmoid(a) * b).astype(_BF)
    part = jnp.dot(h, sw2_ref[...].astype(_BF), preferred_element_type=_F32)

    @pl.when(j == 0)
    def _():
        acc_ref[...] = part

    @pl.when(j != 0)
    def _():
        acc_ref[...] = acc_ref[...] + part

    @pl.when(j == nj - 1)
    def _():
        sg = jnp.dot(xb, sgw_ref[...].astype(_BF), preferred_element_type=_F32)
        out_ref[...] = routed_ref[...] + acc_ref[...] * jax.nn.sigmoid(sg)


def kernel(hidden_states, gate_w, w1, w3, w2, sw1, sw3, sw2, shared_gate_w):
    orig_shape = hidden_states.shape
    x = hidden_states.reshape(-1, D)

    gates, rank = pl.pallas_call(
        _routing_kernel,
        out_shape=(
            jax.ShapeDtypeStruct((T, E), _F32),
            jax.ShapeDtypeStruct((T, E), _F32),
        ),
    )(x, gate_w)

    xb = x.astype(_BF)

    routed = pl.pallas_call(
        _expert_kernel,
        grid=(E,),
        in_specs=[
            pl.BlockSpec((T, D), lambda e: (0, 0)),
            pl.BlockSpec((T, E), lambda e: (0, 0)),
            pl.BlockSpec((T, E), lambda e: (0, 0)),
            pl.BlockSpec((1, D, F_MOE), lambda e: (e, 0, 0)),
            pl.BlockSpec((1, D, F_MOE), lambda e: (e, 0, 0)),
            pl.BlockSpec((1, F_MOE, D), lambda e: (e, 0, 0)),
        ],
        out_specs=pl.BlockSpec((T, D), lambda e: (0, 0)),
        out_shape=jax.ShapeDtypeStruct((T, D), _F32),
        scratch_shapes=[pltpu.VMEM((2, CAP, D), _BF)],
    )(xb, gates, rank, w1, w3, w2)

    tbs = 512
    out = pl.pallas_call(
        _shared_kernel,
        grid=(T // tbs, 2),
        in_specs=[
            pl.BlockSpec((tbs, D), lambda t, j: (t, 0)),
            pl.BlockSpec((tbs, D), lambda t, j: (t, 0)),
            pl.BlockSpec((D, F_SH // 2), lambda t, j: (0, j)),
            pl.BlockSpec((D, F_SH // 2), lambda t, j: (0, j)),
            pl.BlockSpec((F_SH // 2, D), lambda t, j: (j, 0)),
            pl.BlockSpec((D, 1), lambda t, j: (0, 0)),
        ],
        out_specs=pl.BlockSpec((tbs, D), lambda t, j: (t, 0)),
        out_shape=jax.ShapeDtypeStruct((T, D), _F32),
        scratch_shapes=[pltpu.VMEM((tbs, D), _F32)],
    )(xb, routed, sw1, sw3, sw2, shared_gate_w)

    return out.reshape(orig_shape)


# speedup vs baseline: 1.2441x; 1.2441x over previous
import jax
import jax.numpy as jnp
from jax.experimental import pallas as pl
from jax.experimental.pallas import tpu as pltpu

T = 2048
D = 1024
E = 64
K = 8
F_MOE = 1408
F_SH = 2816
CAP = 512

_BF = jnp.bfloat16
_F32 = jnp.float32

_TCH = 512
_FCH = (768, 640)


def _routing_kernel(x_ref, gw_ref, gates_ref, rank_ref, bnd_ref):
    x = x_ref[...]
    logits = jnp.dot(x, gw_ref[...], precision=jax.lax.Precision.HIGHEST)
    m = jnp.max(logits, axis=1, keepdims=True)
    p = jnp.exp(logits - m)
    p = p / jnp.sum(p, axis=1, keepdims=True)

    work = p
    kth = None
    for _ in range(K):
        kth = jnp.max(work, axis=1, keepdims=True)
        work = jnp.where(work == kth, -1.0, work)
    gates = jnp.where(p >= kth, p, 0.0)

    maskb = (gates > 0.0).astype(_BF)
    blk = 256
    row_i = jax.lax.broadcasted_iota(jnp.int32, (blk, blk), 0)
    col_i = jax.lax.broadcasted_iota(jnp.int32, (blk, blk), 1)
    ltri = (col_i <= row_i).astype(_BF)
    carry = jnp.zeros((1, E), dtype=_F32)
    chunks = []
    bnds = [carry]
    for c in range(T // blk):
        mc = maskb[c * blk:(c + 1) * blk, :]
        rc = jnp.dot(ltri, mc, preferred_element_type=_F32) + carry
        carry = carry + jnp.sum(mc.astype(_F32), axis=0, keepdims=True)
        chunks.append(rc)
        if (c + 1) % (_TCH // blk) == 0:
            bnds.append(carry)
    rank = jnp.concatenate(chunks, axis=0)
    rank = jnp.where(gates > 0.0, rank, 0.0)

    gates_ref[...] = jnp.transpose(gates)
    rank_ref[...] = jnp.transpose(rank)
    nb = len(bnds)
    bnds += [carry] * (8 - nb)
    bnd_ref[...] = jnp.concatenate(bnds, axis=0)


_CB = 128
_W = 256


def _expert_kernel(bnd_ref, xb_ref, gates_ref, rank_ref, w1_ref, w3_ref,
                   w2_ref, out_ref, xe_ref, yb_ref):
    e = pl.program_id(0)
    g_row = gates_ref[0]
    r_row = rank_ref[0].astype(jnp.int32)

    nt = T // _TCH
    lohi = [bnd_ref[i, e].astype(jnp.int32) for i in range(nt + 1)]
    offs = [jnp.minimum((lohi[tb] // _CB) * _CB, CAP - _W)
            for tb in range(nt)]
    w_iota = jax.lax.broadcasted_iota(jnp.int32, (_W, 1), 0)
    top_i = w_iota + (CAP - _W + 1)

    xe_ref[...] = jnp.zeros((CAP, D), dtype=_F32)
    for tb in range(nt):
        sl = slice(tb * _TCH, (tb + 1) * _TCH)
        lo, hi, off = lohi[tb], lohi[tb + 1], offs[tb]

        @pl.when(hi > lo)
        def _(sl=sl, off=off):
            ptct = (r_row[:, sl] == w_iota + (off + 1)).astype(_BF)
            xe_ref[pl.ds(off, _W), :] = xe_ref[pl.ds(off, _W), :] + jnp.dot(
                ptct, xb_ref[sl], preferred_element_type=_F32)

        @pl.when(hi > off + _W)
        def _(sl=sl, off=off):
            eq = (r_row[:, sl] == top_i) & (top_i > off + _W)
            ptct = eq.astype(_BF)
            xe_ref[CAP - _W:, :] = xe_ref[CAP - _W:, :] + jnp.dot(
                ptct, xb_ref[sl], preferred_element_type=_F32)
    xe = xe_ref[...].astype(_BF)

    y = jnp.zeros((CAP, D), dtype=_F32)
    lo = 0
    for sz in _FCH:
        w1h = w1_ref[0, :, lo:lo + sz].astype(_BF)
        a = jnp.dot(xe, w1h, preferred_element_type=_F32)
        w3h = w3_ref[0, :, lo:lo + sz].astype(_BF)
        b = jnp.dot(xe, w3h, preferred_element_type=_F32)
        h = (a * jax.nn.sigmoid(a) * b).astype(_BF)
        w2h = w2_ref[0, lo:lo + sz, :].astype(_BF)
        y = y + jnp.dot(h, w2h, preferred_element_type=_F32)
        lo += sz
    yb_ref[...] = y.astype(_BF)

    @pl.when(e == 0)
    def _():
        out_ref[...] = jnp.zeros((T, D), dtype=_F32)

    for tb in range(nt):
        sl = slice(tb * _TCH, (tb + 1) * _TCH)
        lo, hi, off = lohi[tb], lohi[tb + 1], offs[tb]

        @pl.when(hi > lo)
        def _(sl=sl, off=off):
            eq = r_row[:, sl] == w_iota + (off + 1)
            ptct_g = jnp.where(eq, g_row[:, sl], 0.0).astype(_BF)
            contrib = jax.lax.dot_general(
                ptct_g, yb_ref[pl.ds(off, _W), :], (((0,), (0,)), ((), ())),
                preferred_element_type=_F32)
            out_ref[sl] = out_ref[sl] + contrib

        @pl.when(hi > off + _W)
        def _(sl=sl, off=off):
            eq = (r_row[:, sl] == top_i) & (top_i > off + _W)
            ptct_g = jnp.where(eq, g_row[:, sl], 0.0).astype(_BF)
            contrib = jax.lax.dot_general(
                ptct_g, yb_ref[CAP - _W:, :], (((0,), (0,)), ((), ())),
                preferred_element_type=_F32)
            out_ref[sl] = out_ref[sl] + contrib


def _shared_kernel(xb_ref, routed_ref, sw1_ref, sw3_ref, sw2_ref, sgw_ref,
                   out_ref, acc_ref):
    j = pl.program_id(0)
    t = pl.program_id(1)
    nj = pl.num_programs(0)
    tbs = T // pl.num_programs(1)
    sl = pl.ds(t * tbs, tbs)
    xb = xb_ref[...]
    a = jnp.dot(xb, sw1_ref[...].astype(_BF), preferred_element_type=_F32)
    b = jnp.dot(xb, sw3_ref[...].astype(_BF), preferred_element_type=_F32)
    h = (a * jax.nn.sigmoid(a) * b).astype(_BF)
    part = jnp.dot(h, sw2_ref[...].astype(_BF), preferred_element_type=_F32)

    @pl.when(j == 0)
    def _():
        acc_ref[sl, :] = part

    @pl.when(j != 0)
    def _():
        acc_ref[sl, :] = acc_ref[sl, :] + part

    @pl.when(j == nj - 1)
    def _():
        sg = jnp.dot(xb, sgw_ref[...].astype(_BF), preferred_element_type=_F32)
        out_ref[...] = routed_ref[...] + acc_ref[sl, :] * jax.nn.sigmoid(sg)


def kernel(hidden_states, gate_w, w1, w3, w2, sw1, sw3, sw2, shared_gate_w):
    orig_shape = hidden_states.shape
    x = hidden_states.reshape(-1, D)

    gates_c, rank_c, bnd = pl.pallas_call(
        _routing_kernel,
        out_shape=(
            jax.ShapeDtypeStruct((E, T), _F32),
            jax.ShapeDtypeStruct((E, T), _F32),
            jax.ShapeDtypeStruct((8, E), _F32),
        ),
    )(x, gate_w)
    gates_c = gates_c.reshape(E, 1, T)
    rank_c = rank_c.reshape(E, 1, T)

    xb = x.astype(_BF)

    routed = pl.pallas_call(
        _expert_kernel,
        grid=(E,),
        in_specs=[
            pl.BlockSpec(memory_space=pltpu.SMEM),
            pl.BlockSpec((T, D), lambda e: (0, 0)),
            pl.BlockSpec((1, 1, T), lambda e: (e, 0, 0)),
            pl.BlockSpec((1, 1, T), lambda e: (e, 0, 0)),
            pl.BlockSpec((1, D, F_MOE), lambda e: (e, 0, 0)),
            pl.BlockSpec((1, D, F_MOE), lambda e: (e, 0, 0)),
            pl.BlockSpec((1, F_MOE, D), lambda e: (e, 0, 0)),
        ],
        out_specs=pl.BlockSpec((T, D), lambda e: (0, 0)),
        out_shape=jax.ShapeDtypeStruct((T, D), _F32),
        scratch_shapes=[pltpu.VMEM((CAP, D), _F32),
                        pltpu.VMEM((CAP, D), _BF)],
    )(bnd, xb, gates_c, rank_c, w1, w3, w2)

    tbs = 512
    out = pl.pallas_call(
        _shared_kernel,
        grid=(2, T // tbs),
        in_specs=[
            pl.BlockSpec((tbs, D), lambda j, t: (t, 0)),
            pl.BlockSpec((tbs, D), lambda j, t: (t, 0)),
            pl.BlockSpec((D, F_SH // 2), lambda j, t: (0, j)),
            pl.BlockSpec((D, F_SH // 2), lambda j, t: (0, j)),
            pl.BlockSpec((F_SH // 2, D), lambda j, t: (j, 0)),
            pl.BlockSpec((D, 1), lambda j, t: (0, 0)),
        ],
        out_specs=pl.BlockSpec((tbs, D), lambda j, t: (t, 0)),
        out_shape=jax.ShapeDtypeStruct((T, D), _F32),
        scratch_shapes=[pltpu.VMEM((T, D), _F32)],
    )(xb, routed, sw1, sw3, sw2, shared_gate_w)

    return out.reshape(orig_shape)
